# baseline (speedup 1.0000x reference)
import jax
import jax.numpy as jnp
import numpy as np
from jax import lax
from jax.experimental import pallas as pl
from jax.experimental.pallas import tpu as pltpu
from jax.experimental.pallas import tpu_sc as plsc

N = 100000
E = 3200000
C = 1024
KB = C // 128
NCHUNK = E // C
NWORK = 32
CH_LO = NCHUNK // NWORK
CH_EXTRA = NCHUNK - CH_LO * NWORK
NPAIR = (CH_LO + 2) // 2
RBLK = 2000
NRB = N // RBLK
INV_SQRT3 = 1.0 / np.sqrt(3.0)


def _sc_body(h_hbm, sh_hbm, ei_hbm, w_hbm, z_hbm, out_hbm,
             ei_v, sh_v, hrow_v, msg_v, dst_v, w_v, acc,
             gsem0, gsem1, asem0, asem1):
    cid = lax.axis_index("c")
    sid = lax.axis_index("s")
    wid = sid * 2 + cid

    gsem = [gsem0, gsem1]
    asem = [asem0, asem1]

    pltpu.sync_copy(w_hbm, w_v)
    for i in range(-(-NRB // 16)):
        blk = sid + i * 16

        @pl.when(blk < NRB)
        def _():
            pltpu.sync_copy(z_hbm.at[pl.ds(blk * RBLK, RBLK)],
                            acc.at[pl.ds(blk * RBLK, RBLK)])

    plsc.subcore_barrier()

    W = [w_v[r] for r in range(16)]
    iota = lax.iota(jnp.int32, 16)
    cols = [jnp.full((16,), c, jnp.int32) for c in range(8)]

    start = wid * CH_LO + jnp.minimum(wid, CH_EXTRA)
    nch = CH_LO + jnp.where(wid < CH_EXTRA, 1, 0)

    def prefetch(s, k):
        off = (start + k) * KB
        pltpu.sync_copy(ei_hbm.at[pl.ds(off, KB)], ei_v.at[s])
        pltpu.async_copy(sh_hbm.at[pl.ds(off, KB)], sh_v.at[s], gsem[s])
        for j in range(KB):
            pltpu.async_copy(h_hbm.at[ei_v.at[s, j, 0]],
                             hrow_v.at[s, pl.ds(j * 128, 128)], gsem[s])

    def wait_gathers(s):
        pltpu.make_async_copy(
            sh_hbm.at[pl.ds(0, KB)], sh_v.at[s], gsem[s]).wait()
        for j in range(KB):
            pltpu.make_async_copy(
                h_hbm.at[ei_v.at[s, j, 0]],
                hrow_v.at[s, pl.ds(j * 128, 128)], gsem[s]).wait()

    def fire_adds(s):
        for j in range(KB):
            pltpu.async_copy(msg_v.at[s, pl.ds(j * 128, 128)],
                             acc.at[dst_v.at[s, j]], asem[s], add=True)

    def wait_adds(s):
        for j in range(KB):
            pltpu.make_async_copy(
                msg_v.at[s, pl.ds(j * 128, 128)],
                acc.at[dst_v.at[s, j]], asem[s]).wait()

    def compute(s):
        def step(t, carry):
            j = t >> 3
            c16 = (t & 7) * 16
            ridx = t * 16 + iota
            dst_v[s, j, pl.ds(c16, 16)] = ei_v[s, j, 1, pl.ds(c16, 16)]
            hcol = lambda c: plsc.load_gather(
                hrow_v.at[s], [ridx, cols[c]])
            scol = lambda c: sh_v[s, j, c, pl.ds(c16, 16)]
            s0, s1 = hcol(0), hcol(1)
            v00, v01, v02 = hcol(2), hcol(3), hcol(4)
            v10, v11, v12 = hcol(5), hcol(6), hcol(7)
            e0, e1x, e1y, e1z = scol(0), scol(1), scol(2), scol(3)
            p0 = s0 * e0
            p1 = s1 * e0
            d0 = v00 * e1x + v01 * e1y + v02 * e1z
            d1 = v10 * e1x + v11 * e1y + v12 * e1z
            m0 = W[0] * p0 + W[1] * p1 + W[4] * d0 + W[5] * d1
            m1 = W[2] * p0 + W[3] * p1 + W[6] * d0 + W[7] * d1
            q0 = W[8] * s0 + W[9] * s1
            q1 = W[10] * s0 + W[11] * s1
            m2 = q0 * e1x + e0 * (W[12] * v00 + W[13] * v10)
            m3 = q0 * e1y + e0 * (W[12] * v01 + W[13] * v11)
            m4 = q0 * e1z + e0 * (W[12] * v02 + W[13] * v12)
            m5 = q1 * e1x + e0 * (W[14] * v00 + W[15] * v10)
            m6 = q1 * e1y + e0 * (W[14] * v01 + W[15] * v11)
            m7 = q1 * e1z + e0 * (W[14] * v02 + W[15] * v12)
            for c, m in enumerate((m0, m1, m2, m3, m4, m5, m6, m7)):
                plsc.store_scatter(msg_v.at[s], [ridx, cols[c]], m)
            return carry

        lax.fori_loop(0, C // 16, step, 0)

    def stage(s, k, i):
        valid = k < nch

        @pl.when(valid)
        def _():
            wait_gathers(s)

        @pl.when((i > 0) & valid)
        def _():
            wait_adds(s)

        @pl.when(valid)
        def _():
            compute(s)
            fire_adds(s)

        @pl.when(k + 2 < nch)
        def _():
            prefetch(s, k + 2)

    prefetch(0, 0)
    prefetch(1, 1)

    def body(i, carry):
        stage(0, i * 2, i)
        stage(1, i * 2 + 1, i)
        return carry

    lax.fori_loop(0, NPAIR, body, 0)

    wait_adds(0)
    wait_adds(1)

    plsc.subcore_barrier()
    for i in range(-(-NRB // 16)):
        blk = sid + i * 16

        @pl.when(blk < NRB)
        def _():
            pltpu.sync_copy(acc.at[pl.ds(blk * RBLK, RBLK)],
                            out_hbm.at[cid, pl.ds(blk * RBLK, RBLK)])


def _add_body(x_ref, o_ref):
    o_ref[...] = x_ref[0] + x_ref[1]


@jax.jit
def kernel(h, sh, edge_index, w1, w2, w3, w4):
    a1 = 0.5 * w1[:, 0, :]
    a4 = (0.5 * INV_SQRT3) * w4[:, 0, :]
    a2 = 0.5 * w2[:, 0, :]
    a3 = 0.5 * w3[:, 0, :]
    wflat = jnp.concatenate(
        [a1.T.ravel(), a4.T.ravel(), a2.T.ravel(), a3.T.ravel()])
    wsplat = jnp.tile(wflat[:, None], (1, 16))

    ei3 = edge_index.reshape(2, E // 128, 128).transpose(1, 0, 2)
    sh3 = sh.T.reshape(4, E // 128, 128).transpose(1, 0, 2)
    zeros = jnp.zeros((N, 8), jnp.float32)

    mesh = plsc.VectorSubcoreMesh(core_axis_name="c", subcore_axis_name="s")
    sc = pl.kernel(
        _sc_body,
        out_type=jax.ShapeDtypeStruct((2, N, 8), jnp.float32),
        mesh=mesh,
        scratch_types=[
            pltpu.VMEM((2, KB, 2, 128), jnp.int32),
            pltpu.VMEM((2, KB, 4, 128), jnp.float32),
            pltpu.VMEM((2, C, 8), jnp.float32),
            pltpu.VMEM((2, C, 8), jnp.float32),
            pltpu.VMEM((2, KB, 128), jnp.int32),
            pltpu.VMEM((16, 16), jnp.float32),
            pltpu.VMEM_SHARED((N, 8), jnp.float32),
            pltpu.SemaphoreType.DMA,
            pltpu.SemaphoreType.DMA,
            pltpu.SemaphoreType.DMA,
            pltpu.SemaphoreType.DMA,
        ],
        compiler_params=pltpu.CompilerParams(
            needs_layout_passes=False, use_tc_tiling_on_sc=False),
    )
    partials = sc(h, sh3, ei3, wsplat, zeros)

    out = pl.pallas_call(
        _add_body,
        grid=(N // RBLK,),
        in_specs=[pl.BlockSpec((2, RBLK, 8), lambda i: (0, i, 0))],
        out_specs=pl.BlockSpec((RBLK, 8), lambda i: (i, 0)),
        out_shape=jax.ShapeDtypeStruct((N, 8), jnp.float32),
    )(partials)
    return out


# speedup vs baseline: 1.1896x; 1.1896x over previous
import jax
import jax.numpy as jnp
import numpy as np
from jax import lax
from jax.experimental import pallas as pl
from jax.experimental.pallas import tpu as pltpu
from jax.experimental.pallas import tpu_sc as plsc

N = 100000
E = 3200000
C = 1024
KB = C // 128
NCHUNK = E // C
NWORK = 32
CH_LO = NCHUNK // NWORK
CH_EXTRA = NCHUNK - CH_LO * NWORK
NPAIR = (CH_LO + 2) // 2
RBLK = 2000
NRB = N // RBLK
INV_SQRT3 = 1.0 / np.sqrt(3.0)


def _sc_body(h_hbm, sh_hbm, ei_hbm, w_hbm, z_hbm, out_hbm,
             ei_v, sh_v, hrow_v, msg_v, dst_v, w_v, acc,
             gsem0, gsem1, asem0, asem1):
    cid = lax.axis_index("c")
    sid = lax.axis_index("s")
    wid = sid * 2 + cid

    gsem = [gsem0, gsem1]
    asem = [asem0, asem1]

    pltpu.sync_copy(w_hbm, w_v)
    for i in range(-(-NRB // 16)):
        blk = sid + i * 16

        @pl.when(blk < NRB)
        def _():
            pltpu.sync_copy(z_hbm.at[pl.ds(blk * RBLK, RBLK)],
                            acc.at[pl.ds(blk * RBLK, RBLK)])

    plsc.subcore_barrier()

    W = [w_v[r] for r in range(16)]
    iota = lax.iota(jnp.int32, 16)
    cols = [jnp.full((16,), c, jnp.int32) for c in range(8)]

    start = wid * CH_LO + jnp.minimum(wid, CH_EXTRA)
    nch = CH_LO + jnp.where(wid < CH_EXTRA, 1, 0)

    def prefetch(s, k):
        off = (start + k) * KB
        pltpu.sync_copy(ei_hbm.at[pl.ds(off, KB)], ei_v.at[s])
        pltpu.async_copy(sh_hbm.at[pl.ds(off, KB)], sh_v.at[s], gsem[s])
        for j in range(KB):
            pltpu.async_copy(h_hbm.at[ei_v.at[s, j, 0]],
                             hrow_v.at[s, pl.ds(j * 128, 128)], gsem[s])

    def wait_gathers(s):
        pltpu.make_async_copy(
            sh_hbm.at[pl.ds(0, KB)], sh_v.at[s], gsem[s]).wait()
        for j in range(KB):
            pltpu.make_async_copy(
                h_hbm.at[ei_v.at[s, j, 0]],
                hrow_v.at[s, pl.ds(j * 128, 128)], gsem[s]).wait()

    def fire_adds(s):
        for j in range(KB):
            pltpu.async_copy(msg_v.at[s, pl.ds(j * 128, 128)],
                             acc.at[dst_v.at[s, j]], asem[s], add=True)

    def wait_adds(s):
        for j in range(KB):
            pltpu.make_async_copy(
                msg_v.at[s, pl.ds(j * 128, 128)],
                acc.at[dst_v.at[s, j]], asem[s]).wait()

    def compute(s):
        def step(t, carry):
            j = t >> 3
            c16 = (t & 7) * 16
            ridx = t * 16 + iota
            dst_v[s, j, pl.ds(c16, 16)] = ei_v[s, j, 1, pl.ds(c16, 16)]
            hcol = lambda c: plsc.load_gather(
                hrow_v.at[s], [ridx, cols[c]])
            scol = lambda c: sh_v[s, j, c, pl.ds(c16, 16)]
            s0, s1 = hcol(0), hcol(1)
            v00, v01, v02 = hcol(2), hcol(3), hcol(4)
            v10, v11, v12 = hcol(5), hcol(6), hcol(7)
            e0, e1x, e1y, e1z = scol(0), scol(1), scol(2), scol(3)
            p0 = s0 * e0
            p1 = s1 * e0
            d0 = v00 * e1x + v01 * e1y + v02 * e1z
            d1 = v10 * e1x + v11 * e1y + v12 * e1z
            m0 = W[0] * p0 + W[1] * p1 + W[4] * d0 + W[5] * d1
            m1 = W[2] * p0 + W[3] * p1 + W[6] * d0 + W[7] * d1
            q0 = W[8] * s0 + W[9] * s1
            q1 = W[10] * s0 + W[11] * s1
            m2 = q0 * e1x + e0 * (W[12] * v00 + W[13] * v10)
            m3 = q0 * e1y + e0 * (W[12] * v01 + W[13] * v11)
            m4 = q0 * e1z + e0 * (W[12] * v02 + W[13] * v12)
            m5 = q1 * e1x + e0 * (W[14] * v00 + W[15] * v10)
            m6 = q1 * e1y + e0 * (W[14] * v01 + W[15] * v11)
            m7 = q1 * e1z + e0 * (W[14] * v02 + W[15] * v12)
            for c, m in enumerate((m0, m1, m2, m3, m4, m5, m6, m7)):
                plsc.store_scatter(msg_v.at[s], [ridx, cols[c]], m)
            return carry

        lax.fori_loop(0, C // 16, step, 0)

    def stage(s, k, i):
        valid = k < nch

        @pl.when(valid)
        def _():
            wait_gathers(s)

        @pl.when((i > 0) & valid)
        def _():
            wait_adds(s)

        @pl.when(valid)
        def _():
            compute(s)
            fire_adds(s)

        @pl.when(k + 2 < nch)
        def _():
            prefetch(s, k + 2)

    prefetch(0, 0)
    prefetch(1, 1)

    def body(i, carry):
        stage(0, i * 2, i)
        stage(1, i * 2 + 1, i)
        return carry

    lax.fori_loop(0, NPAIR, body, 0)

    wait_adds(0)
    wait_adds(1)

    plsc.subcore_barrier()
    for i in range(-(-NRB // 16)):
        blk = sid + i * 16

        @pl.when(blk < NRB)
        def _():
            pltpu.sync_copy(acc.at[pl.ds(blk * RBLK, RBLK)],
                            out_hbm.at[cid, pl.ds(blk * RBLK, RBLK)])


def _add_body(x_ref, o_ref):
    o_ref[...] = x_ref[0] + x_ref[1]


@jax.jit
def kernel(h, sh, edge_index, w1, w2, w3, w4):
    a1 = 0.5 * w1[:, 0, :]
    a4 = (0.5 * INV_SQRT3) * w4[:, 0, :]
    a2 = 0.5 * w2[:, 0, :]
    a3 = 0.5 * w3[:, 0, :]
    wflat = jnp.concatenate(
        [a1.T.ravel(), a4.T.ravel(), a2.T.ravel(), a3.T.ravel()])
    wsplat = jnp.tile(wflat[:, None], (1, 16))

    ei3 = edge_index.reshape(2, E // 128, 128).transpose(1, 0, 2)
    sh3 = sh.T.reshape(4, E // 128, 128).transpose(1, 0, 2)
    zeros = jnp.zeros((N, 8), jnp.float32)

    mesh = plsc.VectorSubcoreMesh(core_axis_name="c", subcore_axis_name="s")
    sc = pl.kernel(
        _sc_body,
        out_type=jax.ShapeDtypeStruct((2, N, 8), jnp.float32),
        mesh=mesh,
        scratch_types=[
            pltpu.VMEM((2, KB, 2, 128), jnp.int32),
            pltpu.VMEM((2, KB, 4, 128), jnp.float32),
            pltpu.VMEM((2, C, 8), jnp.float32),
            pltpu.VMEM((2, C, 8), jnp.float32),
            pltpu.VMEM((2, KB, 128), jnp.int32),
            pltpu.VMEM((16, 16), jnp.float32),
            pltpu.VMEM_SHARED((N, 8), jnp.float32),
            pltpu.SemaphoreType.DMA,
            pltpu.SemaphoreType.DMA,
            pltpu.SemaphoreType.DMA,
            pltpu.SemaphoreType.DMA,
        ],
        compiler_params=pltpu.CompilerParams(
            needs_layout_passes=False, use_tc_tiling_on_sc=False),
    )
    partials = sc(h, sh3, ei3, wsplat, zeros)

    x = partials.reshape(2, (N * 8) // 128, 128)
    out = pl.pallas_call(
        _add_body,
        out_shape=jax.ShapeDtypeStruct(((N * 8) // 128, 128), jnp.float32),
    )(x)
    return out.reshape(N, 8)
